# baseline (speedup 1.0000x reference)
import jax
import jax.numpy as jnp
from jax import lax
from jax.experimental import pallas as pl
from jax.experimental.pallas import tpu as pltpu
from jax.experimental.pallas import tpu_sc as plsc

N = 2048
D = 512
E = 65536
P = 8
TOPK = 10
HID = 512
OUT = 16
ALPHA = 0.5
K = P * D

BM = 256
NBLK = N // BM

SC_TILES = 32
ROWS = 32
PASSES = N // (SC_TILES * ROWS)
CH = 16384
NCHUNK = E // CH
GRP = CH // 16


def _prep_body(x_ref, pw_ref, w0_ref, y_ref, yt_ref, xw0_ref):
    xb = x_ref[...]
    for p in range(P):
        t = xb * pw_ref[p, :][None, :]
        s2 = jnp.sum(t * t, axis=1, keepdims=True)
        yp = (t / jnp.maximum(jnp.sqrt(s2), 1e-12)).astype(jnp.bfloat16)
        y_ref[:, p * D:(p + 1) * D] = yp
        yt_ref[p * D:(p + 1) * D, :] = yp.T
    xw0_ref[...] = lax.dot_general(
        xb, w0_ref[...], (((1,), (0,)), ((), ())),
        preferred_element_type=jnp.float32, precision=lax.Precision.HIGHEST)


def _main_body(yblk_ref, yt_ref, adjA_ref):
    att = lax.dot_general(
        yblk_ref[...], yt_ref[...], (((1,), (0,)), ((), ())),
        preferred_element_type=jnp.float32) * (1.0 / P)
    t = jnp.full((BM, 1), jnp.inf, jnp.float32)
    c = jnp.zeros((BM, 1), jnp.float32)
    for _ in range(TOPK):
        work = jnp.where(att < t, att, -jnp.inf)
        m = jnp.max(work, axis=1, keepdims=True)
        cnt = jnp.sum(jnp.where(att == m, 1.0, 0.0), axis=1, keepdims=True)
        upd = c < TOPK
        t = jnp.where(upd, m, t)
        c = jnp.where(upd, c + cnt, c)
    adj_top = jnp.where(att >= t, att, 0.0)
    rs = jnp.sum(adj_top, axis=1, keepdims=True)
    adjA_ref[...] = (adj_top / jnp.clip(rs, 1e-12, 1.0)) * ALPHA


def _finish_body(adjA_ref, adji_ref, xw0_ref, b0_ref, w1_ref, b1_ref,
                 h1_ref, out_ref, adj_scr, hw_scr):
    i = pl.program_id(0)
    adjb = adjA_ref[...] + adji_ref[...]
    adj_scr[pl.ds(i * BM, BM), :] = adjb
    h1 = jnp.maximum(
        lax.dot_general(adjb, xw0_ref[...], (((1,), (0,)), ((), ())),
                        preferred_element_type=jnp.float32,
                        precision=lax.Precision.HIGHEST) + b0_ref[...], 0.0)
    h1_ref[...] = h1
    hw_scr[pl.ds(i * BM, BM), :] = lax.dot_general(
        h1, w1_ref[...], (((1,), (0,)), ((), ())),
        preferred_element_type=jnp.float32,
        precision=lax.Precision.HIGHEST)

    @pl.when(i == NBLK - 1)
    def _():
        out_ref[...] = lax.dot_general(
            adj_scr[...], hw_scr[...], (((1,), (0,)), ((), ())),
            preferred_element_type=jnp.float32,
            precision=lax.Precision.HIGHEST) + b1_ref[...]


def _adj_ini_sc(src_hbm, dst_hbm, zrow_hbm, out_hbm, accum, sbuf, dbuf):
    wid = lax.axis_index("s") * 2 + lax.axis_index("c")
    ones = jnp.full((16,), 1.0, jnp.float32)
    for pz in range(PASSES):
        row_base = pz * (SC_TILES * ROWS) + wid * ROWS
        pltpu.sync_copy(zrow_hbm, accum)
        def chunk_body(ci, _):
            c0 = pl.multiple_of(ci * CH, 8)
            pltpu.sync_copy(src_hbm.at[pl.ds(c0, CH)], sbuf)
            pltpu.sync_copy(dst_hbm.at[pl.ds(c0, CH)], dbuf)
            def grp_body(j, carry):
                s = sbuf[pl.ds(j * 16, 16)]
                d = dbuf[pl.ds(j * 16, 16)]
                rel = s - row_base
                msk = rel.astype(jnp.uint32) < jnp.uint32(ROWS)
                plsc.addupdate_scatter(accum, [rel, d], ones, mask=msk)
                return carry
            lax.fori_loop(0, GRP, grp_body, 0)
            return _
        lax.fori_loop(0, NCHUNK, chunk_body, 0)
        pltpu.sync_copy(accum, out_hbm.at[pl.ds(row_base, ROWS), :])


def _build_adj_ini(src, dst, zrow):
    mesh = plsc.VectorSubcoreMesh(core_axis_name="c", subcore_axis_name="s")
    return pl.kernel(
        _adj_ini_sc,
        mesh=mesh,
        compiler_params=pltpu.CompilerParams(needs_layout_passes=False),
        out_type=jax.ShapeDtypeStruct((N, N), jnp.float32),
        scratch_types=[
            pltpu.VMEM((ROWS, N), jnp.float32),
            pltpu.VMEM((CH,), jnp.int32),
            pltpu.VMEM((CH,), jnp.int32),
        ],
    )(src, dst, zrow)


_prep_call = pl.pallas_call(
    _prep_body,
    grid=(NBLK,),
    in_specs=[
        pl.BlockSpec((BM, D), lambda i: (i, 0)),
        pl.BlockSpec((P, D), lambda i: (0, 0)),
        pl.BlockSpec((D, HID), lambda i: (0, 0)),
    ],
    out_specs=[
        pl.BlockSpec((BM, K), lambda i: (i, 0)),
        pl.BlockSpec((K, BM), lambda i: (0, i)),
        pl.BlockSpec((BM, HID), lambda i: (i, 0)),
    ],
    out_shape=[
        jax.ShapeDtypeStruct((N, K), jnp.bfloat16),
        jax.ShapeDtypeStruct((K, N), jnp.bfloat16),
        jax.ShapeDtypeStruct((N, HID), jnp.float32),
    ],
)

_main_call = pl.pallas_call(
    _main_body,
    grid=(NBLK,),
    in_specs=[
        pl.BlockSpec((BM, K), lambda i: (i, 0)),
        pl.BlockSpec((K, N), lambda i: (0, 0)),
    ],
    out_specs=pl.BlockSpec((BM, N), lambda i: (i, 0)),
    out_shape=jax.ShapeDtypeStruct((N, N), jnp.float32),
)

_finish_call = pl.pallas_call(
    _finish_body,
    grid=(NBLK,),
    in_specs=[
        pl.BlockSpec((BM, N), lambda i: (i, 0)),
        pl.BlockSpec((BM, N), lambda i: (i, 0)),
        pl.BlockSpec((N, HID), lambda i: (0, 0)),
        pl.BlockSpec((1, HID), lambda i: (0, 0)),
        pl.BlockSpec((HID, OUT), lambda i: (0, 0)),
        pl.BlockSpec((1, OUT), lambda i: (0, 0)),
    ],
    out_specs=[
        pl.BlockSpec((BM, HID), lambda i: (i, 0)),
        pl.BlockSpec((N, OUT), lambda i: (0, 0)),
    ],
    out_shape=[
        jax.ShapeDtypeStruct((N, HID), jnp.float32),
        jax.ShapeDtypeStruct((N, OUT), jnp.float32),
    ],
    scratch_shapes=[
        pltpu.VMEM((N, N), jnp.float32),
        pltpu.VMEM((N, OUT), jnp.float32),
    ],
)


def kernel(x, edge_index, ini_emb, enc_W0, enc_b0, enc_W1, enc_b1, dec_W0,
           dec_b0, dec_W1, dec_b1, pers_weight, gcn_W0, gcn_b0, gcn_W1,
           gcn_b1):
    src = edge_index[0]
    dst = edge_index[1]
    zrow = jnp.zeros((ROWS, N), jnp.float32)
    adji = _build_adj_ini(src, dst, zrow)
    y, yt, xw0 = _prep_call(x, pers_weight, gcn_W0)
    adjA = _main_call(y, yt)
    h1, logits = _finish_call(adjA, adji, xw0, gcn_b0.reshape(1, HID),
                              gcn_W1, gcn_b1.reshape(1, OUT))
    return h1, logits


# speedup vs baseline: 1.0300x; 1.0300x over previous
import jax
import jax.numpy as jnp
from jax import lax
from jax.experimental import pallas as pl
from jax.experimental.pallas import tpu as pltpu
from jax.experimental.pallas import tpu_sc as plsc

N = 2048
D = 512
E = 65536
P = 8
TOPK = 10
HID = 512
OUT = 16
ALPHA = 0.5
K = P * D

BM = 256
NBLK = N // BM

SC_TILES = 32
ROWS = 32
PASSES = N // (SC_TILES * ROWS)
CH = 16384
NCHUNK = E // CH
GRP = CH // 16


def _prep_body(x_ref, pw_ref, w0_ref, y_ref, yt_ref, xw0_ref):
    xb = x_ref[...]
    for p in range(P):
        t = xb * pw_ref[p, :][None, :]
        s2 = jnp.sum(t * t, axis=1, keepdims=True)
        yp = (t / jnp.maximum(jnp.sqrt(s2), 1e-12)).astype(jnp.bfloat16)
        y_ref[:, p * D:(p + 1) * D] = yp
        yt_ref[p * D:(p + 1) * D, :] = yp.T
    xw0_ref[...] = lax.dot_general(
        xb, w0_ref[...], (((1,), (0,)), ((), ())),
        preferred_element_type=jnp.float32, precision=lax.Precision.HIGHEST)


def _main_body(yblk_ref, yt_ref, adjA_ref):
    att = lax.dot_general(
        yblk_ref[...], yt_ref[...], (((1,), (0,)), ((), ())),
        preferred_element_type=jnp.float32) * (1.0 / P)
    t = jnp.full((BM, 1), jnp.inf, jnp.float32)
    c = jnp.zeros((BM, 1), jnp.float32)
    for _ in range(TOPK):
        work = jnp.where(att < t, att, -jnp.inf)
        m = jnp.max(work, axis=1, keepdims=True)
        cnt = jnp.sum(jnp.where(att == m, 1.0, 0.0), axis=1, keepdims=True)
        upd = c < TOPK
        t = jnp.where(upd, m, t)
        c = jnp.where(upd, c + cnt, c)
    adj_top = jnp.where(att >= t, att, 0.0)
    rs = jnp.sum(adj_top, axis=1, keepdims=True)
    adjA_ref[...] = (adj_top / jnp.clip(rs, 1e-12, 1.0)) * ALPHA


def _finish_body(adjA_ref, adji_ref, xw0_ref, b0_ref, w1_ref, b1_ref,
                 h1_ref, out_ref, adj_scr, hw_scr):
    i = pl.program_id(0)
    adjb = adjA_ref[...] + adji_ref[...]
    adj_scr[pl.ds(i * BM, BM), :] = adjb
    h1 = jnp.maximum(
        lax.dot_general(adjb, xw0_ref[...], (((1,), (0,)), ((), ())),
                        preferred_element_type=jnp.float32,
                        precision=lax.Precision.HIGHEST) + b0_ref[...], 0.0)
    h1_ref[...] = h1
    hw_scr[pl.ds(i * BM, BM), :] = lax.dot_general(
        h1, w1_ref[...], (((1,), (0,)), ((), ())),
        preferred_element_type=jnp.float32,
        precision=lax.Precision.HIGHEST)

    @pl.when(i == NBLK - 1)
    def _():
        out_ref[...] = lax.dot_general(
            adj_scr[...], hw_scr[...], (((1,), (0,)), ((), ())),
            preferred_element_type=jnp.float32,
            precision=lax.Precision.HIGHEST) + b1_ref[...]


def _adj_ini_sc(src_hbm, dst_hbm, zrow_hbm, out_hbm, accum, sbuf, dbuf):
    wid = lax.axis_index("s") * 2 + lax.axis_index("c")
    ones = jnp.full((16,), 1.0, jnp.float32)
    for pz in range(PASSES):
        row_base = pz * (SC_TILES * ROWS) + wid * ROWS
        pltpu.sync_copy(zrow_hbm, accum)
        def chunk_body(ci, _):
            c0 = pl.multiple_of(ci * CH, 8)
            pltpu.sync_copy(src_hbm.at[pl.ds(c0, CH)], sbuf)
            pltpu.sync_copy(dst_hbm.at[pl.ds(c0, CH)], dbuf)
            def grp_body(j, carry):
                base = j * 64
                for u in range(4):
                    s = sbuf[pl.ds(base + u * 16, 16)]
                    d = dbuf[pl.ds(base + u * 16, 16)]
                    rel = s - row_base
                    msk = rel.astype(jnp.uint32) < jnp.uint32(ROWS)
                    plsc.addupdate_scatter(accum, [rel, d], ones, mask=msk)
                return carry
            lax.fori_loop(0, GRP // 4, grp_body, 0)
            return _
        lax.fori_loop(0, NCHUNK, chunk_body, 0)
        pltpu.sync_copy(accum, out_hbm.at[pl.ds(row_base, ROWS), :])


def _build_adj_ini(src, dst, zrow):
    mesh = plsc.VectorSubcoreMesh(core_axis_name="c", subcore_axis_name="s")
    return pl.kernel(
        _adj_ini_sc,
        mesh=mesh,
        compiler_params=pltpu.CompilerParams(needs_layout_passes=False),
        out_type=jax.ShapeDtypeStruct((N, N), jnp.float32),
        scratch_types=[
            pltpu.VMEM((ROWS, N), jnp.float32),
            pltpu.VMEM((CH,), jnp.int32),
            pltpu.VMEM((CH,), jnp.int32),
        ],
    )(src, dst, zrow)


_prep_call = pl.pallas_call(
    _prep_body,
    grid=(NBLK,),
    in_specs=[
        pl.BlockSpec((BM, D), lambda i: (i, 0)),
        pl.BlockSpec((P, D), lambda i: (0, 0)),
        pl.BlockSpec((D, HID), lambda i: (0, 0)),
    ],
    out_specs=[
        pl.BlockSpec((BM, K), lambda i: (i, 0)),
        pl.BlockSpec((K, BM), lambda i: (0, i)),
        pl.BlockSpec((BM, HID), lambda i: (i, 0)),
    ],
    out_shape=[
        jax.ShapeDtypeStruct((N, K), jnp.bfloat16),
        jax.ShapeDtypeStruct((K, N), jnp.bfloat16),
        jax.ShapeDtypeStruct((N, HID), jnp.float32),
    ],
)

_main_call = pl.pallas_call(
    _main_body,
    grid=(NBLK,),
    in_specs=[
        pl.BlockSpec((BM, K), lambda i: (i, 0)),
        pl.BlockSpec((K, N), lambda i: (0, 0)),
    ],
    out_specs=pl.BlockSpec((BM, N), lambda i: (i, 0)),
    out_shape=jax.ShapeDtypeStruct((N, N), jnp.float32),
)

_finish_call = pl.pallas_call(
    _finish_body,
    grid=(NBLK,),
    in_specs=[
        pl.BlockSpec((BM, N), lambda i: (i, 0)),
        pl.BlockSpec((BM, N), lambda i: (i, 0)),
        pl.BlockSpec((N, HID), lambda i: (0, 0)),
        pl.BlockSpec((1, HID), lambda i: (0, 0)),
        pl.BlockSpec((HID, OUT), lambda i: (0, 0)),
        pl.BlockSpec((1, OUT), lambda i: (0, 0)),
    ],
    out_specs=[
        pl.BlockSpec((BM, HID), lambda i: (i, 0)),
        pl.BlockSpec((N, OUT), lambda i: (0, 0)),
    ],
    out_shape=[
        jax.ShapeDtypeStruct((N, HID), jnp.float32),
        jax.ShapeDtypeStruct((N, OUT), jnp.float32),
    ],
    scratch_shapes=[
        pltpu.VMEM((N, N), jnp.float32),
        pltpu.VMEM((N, OUT), jnp.float32),
    ],
)


def kernel(x, edge_index, ini_emb, enc_W0, enc_b0, enc_W1, enc_b1, dec_W0,
           dec_b0, dec_W1, dec_b1, pers_weight, gcn_W0, gcn_b0, gcn_W1,
           gcn_b1):
    src = edge_index[0]
    dst = edge_index[1]
    zrow = jnp.zeros((ROWS, N), jnp.float32)
    adji = _build_adj_ini(src, dst, zrow)
    y, yt, xw0 = _prep_call(x, pers_weight, gcn_W0)
    adjA = _main_call(y, yt)
    h1, logits = _finish_call(adjA, adji, xw0, gcn_b0.reshape(1, HID),
                              gcn_W1, gcn_b1.reshape(1, OUT))
    return h1, logits


# speedup vs baseline: 1.1304x; 1.0975x over previous
import jax
import jax.numpy as jnp
from jax import lax
from jax.experimental import pallas as pl
from jax.experimental.pallas import tpu as pltpu
from jax.experimental.pallas import tpu_sc as plsc

N = 2048
D = 512
E = 65536
P = 8
TOPK = 10
HID = 512
OUT = 16
ALPHA = 0.5
K = P * D

BM = 256
NBLK = N // BM

SC_TILES = 32
ROWS = 32
PASSES = N // (SC_TILES * ROWS)
CH = 16384
NCHUNK = E // CH
GRP = CH // 16


def _prep_body(x_ref, pw_ref, w0_ref, y_ref, yt_ref, xw0_ref):
    xb = x_ref[...]
    for p in range(P):
        t = xb * pw_ref[p, :][None, :]
        s2 = jnp.sum(t * t, axis=1, keepdims=True)
        yp = (t / jnp.maximum(jnp.sqrt(s2), 1e-12)).astype(jnp.bfloat16)
        y_ref[:, p * D:(p + 1) * D] = yp
        yt_ref[p * D:(p + 1) * D, :] = yp.T
    xw0_ref[...] = lax.dot_general(
        xb, w0_ref[...], (((1,), (0,)), ((), ())),
        preferred_element_type=jnp.float32, precision=lax.Precision.HIGHEST)


def _main_body(yblk_ref, yt_ref, adjA_ref):
    att = lax.dot_general(
        yblk_ref[...], yt_ref[...], (((1,), (0,)), ((), ())),
        preferred_element_type=jnp.float32) * (1.0 / P)
    t = jnp.full((BM, 1), jnp.inf, jnp.float32)
    c = jnp.zeros((BM, 1), jnp.float32)
    for _ in range(TOPK):
        work = jnp.where(att < t, att, -jnp.inf)
        m = jnp.max(work, axis=1, keepdims=True)
        cnt = jnp.sum(jnp.where(att == m, 1.0, 0.0), axis=1, keepdims=True)
        upd = c < TOPK
        t = jnp.where(upd, m, t)
        c = jnp.where(upd, c + cnt, c)
    adj_top = jnp.where(att >= t, att, 0.0)
    rs = jnp.sum(adj_top, axis=1, keepdims=True)
    adjA_ref[...] = (adj_top / jnp.clip(rs, 1e-12, 1.0)) * ALPHA


def _finish_body(adjA_ref, adji_ref, xw0_ref, b0_ref, w1_ref, b1_ref,
                 h1_ref, out_ref, adj_scr, hw_scr):
    i = pl.program_id(0)
    adjb = adjA_ref[...] + adji_ref[...]
    adj_scr[pl.ds(i * BM, BM), :] = adjb
    h1 = jnp.maximum(
        lax.dot_general(adjb, xw0_ref[...], (((1,), (0,)), ((), ())),
                        preferred_element_type=jnp.float32,
                        precision=lax.Precision.HIGHEST) + b0_ref[...], 0.0)
    h1_ref[...] = h1
    hw_scr[pl.ds(i * BM, BM), :] = lax.dot_general(
        h1, w1_ref[...], (((1,), (0,)), ((), ())),
        preferred_element_type=jnp.float32,
        precision=lax.Precision.HIGHEST)

    @pl.when(i == NBLK - 1)
    def _():
        out_ref[...] = lax.dot_general(
            adj_scr[...], hw_scr[...], (((1,), (0,)), ((), ())),
            preferred_element_type=jnp.float32,
            precision=lax.Precision.HIGHEST) + b1_ref[...]


def _adj_ini_sc(src_hbm, dst_hbm, zrow_hbm, out_hbm, accum, sbuf, dbuf):
    wid = lax.axis_index("s") * 2 + lax.axis_index("c")
    ones = jnp.full((16,), 1.0, jnp.float32)
    for pz in range(PASSES):
        row_base = pz * (SC_TILES * ROWS) + wid * ROWS
        pltpu.sync_copy(zrow_hbm, accum)
        def chunk_body(ci, _):
            c0 = pl.multiple_of(ci * CH, 8)
            pltpu.sync_copy(src_hbm.at[pl.ds(c0, CH)], sbuf)
            pltpu.sync_copy(dst_hbm.at[pl.ds(c0, CH)], dbuf)
            @plsc.parallel_loop(0, GRP, 1, unroll=8)
            def grp_body(j):
                s = sbuf[pl.ds(j * 16, 16)]
                d = dbuf[pl.ds(j * 16, 16)]
                rel = s - row_base
                msk = rel.astype(jnp.uint32) < jnp.uint32(ROWS)
                plsc.addupdate_scatter(accum, [rel, d], ones, mask=msk)
            return _
        lax.fori_loop(0, NCHUNK, chunk_body, 0)
        pltpu.sync_copy(accum, out_hbm.at[pl.ds(row_base, ROWS), :])


def _build_adj_ini(src, dst, zrow):
    mesh = plsc.VectorSubcoreMesh(core_axis_name="c", subcore_axis_name="s")
    return pl.kernel(
        _adj_ini_sc,
        mesh=mesh,
        compiler_params=pltpu.CompilerParams(needs_layout_passes=False),
        out_type=jax.ShapeDtypeStruct((N, N), jnp.float32),
        scratch_types=[
            pltpu.VMEM((ROWS, N), jnp.float32),
            pltpu.VMEM((CH,), jnp.int32),
            pltpu.VMEM((CH,), jnp.int32),
        ],
    )(src, dst, zrow)


_prep_call = pl.pallas_call(
    _prep_body,
    grid=(NBLK,),
    in_specs=[
        pl.BlockSpec((BM, D), lambda i: (i, 0)),
        pl.BlockSpec((P, D), lambda i: (0, 0)),
        pl.BlockSpec((D, HID), lambda i: (0, 0)),
    ],
    out_specs=[
        pl.BlockSpec((BM, K), lambda i: (i, 0)),
        pl.BlockSpec((K, BM), lambda i: (0, i)),
        pl.BlockSpec((BM, HID), lambda i: (i, 0)),
    ],
    out_shape=[
        jax.ShapeDtypeStruct((N, K), jnp.bfloat16),
        jax.ShapeDtypeStruct((K, N), jnp.bfloat16),
        jax.ShapeDtypeStruct((N, HID), jnp.float32),
    ],
)

_main_call = pl.pallas_call(
    _main_body,
    grid=(NBLK,),
    in_specs=[
        pl.BlockSpec((BM, K), lambda i: (i, 0)),
        pl.BlockSpec((K, N), lambda i: (0, 0)),
    ],
    out_specs=pl.BlockSpec((BM, N), lambda i: (i, 0)),
    out_shape=jax.ShapeDtypeStruct((N, N), jnp.float32),
)

_finish_call = pl.pallas_call(
    _finish_body,
    grid=(NBLK,),
    in_specs=[
        pl.BlockSpec((BM, N), lambda i: (i, 0)),
        pl.BlockSpec((BM, N), lambda i: (i, 0)),
        pl.BlockSpec((N, HID), lambda i: (0, 0)),
        pl.BlockSpec((1, HID), lambda i: (0, 0)),
        pl.BlockSpec((HID, OUT), lambda i: (0, 0)),
        pl.BlockSpec((1, OUT), lambda i: (0, 0)),
    ],
    out_specs=[
        pl.BlockSpec((BM, HID), lambda i: (i, 0)),
        pl.BlockSpec((N, OUT), lambda i: (0, 0)),
    ],
    out_shape=[
        jax.ShapeDtypeStruct((N, HID), jnp.float32),
        jax.ShapeDtypeStruct((N, OUT), jnp.float32),
    ],
    scratch_shapes=[
        pltpu.VMEM((N, N), jnp.float32),
        pltpu.VMEM((N, OUT), jnp.float32),
    ],
)


def kernel(x, edge_index, ini_emb, enc_W0, enc_b0, enc_W1, enc_b1, dec_W0,
           dec_b0, dec_W1, dec_b1, pers_weight, gcn_W0, gcn_b0, gcn_W1,
           gcn_b1):
    src = edge_index[0]
    dst = edge_index[1]
    zrow = jnp.zeros((ROWS, N), jnp.float32)
    adji = _build_adj_ini(src, dst, zrow)
    y, yt, xw0 = _prep_call(x, pers_weight, gcn_W0)
    adjA = _main_call(y, yt)
    h1, logits = _finish_call(adjA, adji, xw0, gcn_b0.reshape(1, HID),
                              gcn_W1, gcn_b1.reshape(1, OUT))
    return h1, logits


# speedup vs baseline: 1.1441x; 1.0121x over previous
import jax
import jax.numpy as jnp
from jax import lax
from jax.experimental import pallas as pl
from jax.experimental.pallas import tpu as pltpu
from jax.experimental.pallas import tpu_sc as plsc

N = 2048
D = 512
E = 65536
P = 8
TOPK = 10
HID = 512
OUT = 16
ALPHA = 0.5
K = P * D

BM = 256
NBLK = N // BM

SC_TILES = 32
ROWS = 32
PASSES = N // (SC_TILES * ROWS)
CH = 16384
NCHUNK = E // CH
GRP = CH // 16


def _prep_body(x_ref, pw_ref, w0_ref, y_ref, yt_ref, xw0_ref):
    xb = x_ref[...]
    for p in range(P):
        t = xb * pw_ref[p, :][None, :]
        s2 = jnp.sum(t * t, axis=1, keepdims=True)
        yp = (t / jnp.maximum(jnp.sqrt(s2), 1e-12)).astype(jnp.bfloat16)
        y_ref[:, p * D:(p + 1) * D] = yp
        yt_ref[p * D:(p + 1) * D, :] = yp.T
    xw0_ref[...] = lax.dot_general(
        xb, w0_ref[...], (((1,), (0,)), ((), ())),
        preferred_element_type=jnp.float32, precision=lax.Precision.HIGHEST)


def _main_body(yblk_ref, yt_ref, adjA_ref):
    att = lax.dot_general(
        yblk_ref[...], yt_ref[...], (((1,), (0,)), ((), ())),
        preferred_element_type=jnp.float32) * (1.0 / P)
    t = jnp.full((BM, 1), jnp.inf, jnp.float32)
    c = jnp.zeros((BM, 1), jnp.float32)
    for _ in range(TOPK):
        work = jnp.where(att < t, att, -jnp.inf)
        m = jnp.max(work, axis=1, keepdims=True)
        cnt = jnp.sum(jnp.where(att == m, 1.0, 0.0), axis=1, keepdims=True)
        upd = c < TOPK
        t = jnp.where(upd, m, t)
        c = jnp.where(upd, c + cnt, c)
    adj_top = jnp.where(att >= t, att, 0.0)
    rs = jnp.sum(adj_top, axis=1, keepdims=True)
    adjA_ref[...] = (adj_top / jnp.clip(rs, 1e-12, 1.0)) * ALPHA


def _finish_body(adjA_ref, adji_ref, xw0_ref, b0_ref, w1_ref, b1_ref,
                 h1_ref, out_ref, adj_scr, hw_scr):
    i = pl.program_id(0)
    adjb = adjA_ref[...] + adji_ref[...]
    adj_scr[pl.ds(i * BM, BM), :] = adjb
    h1 = jnp.maximum(
        lax.dot_general(adjb, xw0_ref[...], (((1,), (0,)), ((), ())),
                        preferred_element_type=jnp.float32,
                        precision=lax.Precision.HIGHEST) + b0_ref[...], 0.0)
    h1_ref[...] = h1
    hw_scr[pl.ds(i * BM, BM), :] = lax.dot_general(
        h1, w1_ref[...], (((1,), (0,)), ((), ())),
        preferred_element_type=jnp.float32,
        precision=lax.Precision.HIGHEST)

    @pl.when(i == NBLK - 1)
    def _():
        out_ref[...] = lax.dot_general(
            adj_scr[...], hw_scr[...], (((1,), (0,)), ((), ())),
            preferred_element_type=jnp.float32,
            precision=lax.Precision.HIGHEST) + b1_ref[...]


def _adj_ini_sc(src_hbm, dst_hbm, zrow_hbm, out_hbm, accum, sbuf, dbuf):
    wid = lax.axis_index("s") * 2 + lax.axis_index("c")
    ones = jnp.full((16,), 1.0, jnp.float32)
    for pz in range(PASSES):
        row_base = pz * (SC_TILES * ROWS) + wid * ROWS
        pltpu.sync_copy(zrow_hbm, accum)
        def chunk_body(ci, _):
            c0 = pl.multiple_of(ci * CH, 8)
            pltpu.sync_copy(src_hbm.at[pl.ds(c0, CH)], sbuf)
            pltpu.sync_copy(dst_hbm.at[pl.ds(c0, CH)], dbuf)
            @plsc.parallel_loop(0, GRP, 1, unroll=16)
            def grp_body(j):
                s = sbuf[pl.ds(j * 16, 16)]
                d = dbuf[pl.ds(j * 16, 16)]
                rel = s - row_base
                msk = rel.astype(jnp.uint32) < jnp.uint32(ROWS)
                plsc.addupdate_scatter(accum, [rel, d], ones, mask=msk)
            return _
        lax.fori_loop(0, NCHUNK, chunk_body, 0)
        pltpu.sync_copy(accum, out_hbm.at[pl.ds(row_base, ROWS), :])


def _build_adj_ini(src, dst, zrow):
    mesh = plsc.VectorSubcoreMesh(core_axis_name="c", subcore_axis_name="s")
    return pl.kernel(
        _adj_ini_sc,
        mesh=mesh,
        compiler_params=pltpu.CompilerParams(needs_layout_passes=False),
        out_type=jax.ShapeDtypeStruct((N, N), jnp.float32),
        scratch_types=[
            pltpu.VMEM((ROWS, N), jnp.float32),
            pltpu.VMEM((CH,), jnp.int32),
            pltpu.VMEM((CH,), jnp.int32),
        ],
    )(src, dst, zrow)


_prep_call = pl.pallas_call(
    _prep_body,
    grid=(NBLK,),
    in_specs=[
        pl.BlockSpec((BM, D), lambda i: (i, 0)),
        pl.BlockSpec((P, D), lambda i: (0, 0)),
        pl.BlockSpec((D, HID), lambda i: (0, 0)),
    ],
    out_specs=[
        pl.BlockSpec((BM, K), lambda i: (i, 0)),
        pl.BlockSpec((K, BM), lambda i: (0, i)),
        pl.BlockSpec((BM, HID), lambda i: (i, 0)),
    ],
    out_shape=[
        jax.ShapeDtypeStruct((N, K), jnp.bfloat16),
        jax.ShapeDtypeStruct((K, N), jnp.bfloat16),
        jax.ShapeDtypeStruct((N, HID), jnp.float32),
    ],
)

_main_call = pl.pallas_call(
    _main_body,
    grid=(NBLK,),
    in_specs=[
        pl.BlockSpec((BM, K), lambda i: (i, 0)),
        pl.BlockSpec((K, N), lambda i: (0, 0)),
    ],
    out_specs=pl.BlockSpec((BM, N), lambda i: (i, 0)),
    out_shape=jax.ShapeDtypeStruct((N, N), jnp.float32),
)

_finish_call = pl.pallas_call(
    _finish_body,
    grid=(NBLK,),
    in_specs=[
        pl.BlockSpec((BM, N), lambda i: (i, 0)),
        pl.BlockSpec((BM, N), lambda i: (i, 0)),
        pl.BlockSpec((N, HID), lambda i: (0, 0)),
        pl.BlockSpec((1, HID), lambda i: (0, 0)),
        pl.BlockSpec((HID, OUT), lambda i: (0, 0)),
        pl.BlockSpec((1, OUT), lambda i: (0, 0)),
    ],
    out_specs=[
        pl.BlockSpec((BM, HID), lambda i: (i, 0)),
        pl.BlockSpec((N, OUT), lambda i: (0, 0)),
    ],
    out_shape=[
        jax.ShapeDtypeStruct((N, HID), jnp.float32),
        jax.ShapeDtypeStruct((N, OUT), jnp.float32),
    ],
    scratch_shapes=[
        pltpu.VMEM((N, N), jnp.float32),
        pltpu.VMEM((N, OUT), jnp.float32),
    ],
)


def kernel(x, edge_index, ini_emb, enc_W0, enc_b0, enc_W1, enc_b1, dec_W0,
           dec_b0, dec_W1, dec_b1, pers_weight, gcn_W0, gcn_b0, gcn_W1,
           gcn_b1):
    src = edge_index[0]
    dst = edge_index[1]
    zrow = jnp.zeros((ROWS, N), jnp.float32)
    adji = _build_adj_ini(src, dst, zrow)
    y, yt, xw0 = _prep_call(x, pers_weight, gcn_W0)
    adjA = _main_call(y, yt)
    h1, logits = _finish_call(adjA, adji, xw0, gcn_b0.reshape(1, HID),
                              gcn_W1, gcn_b1.reshape(1, OUT))
    return h1, logits
